# Initial kernel scaffold; baseline (speedup 1.0000x reference)
import jax
import jax.numpy as jnp
from jax.experimental import pallas as pl


def kernel(segin, edgein, segmask, edgemask):
    raise NotImplementedError("write your pallas kernel here")


# speedup vs baseline: 7.7311x; 7.7311x over previous
import functools

import jax
import jax.numpy as jnp
from jax import lax
from jax.experimental import pallas as pl
from jax.experimental.pallas import tpu as pltpu
from jax.experimental.pallas import tpu_sc as plsc

_NC = 19
_IGN = 255
_THR = 0.7
_KEEP = 100000
_B, _H, _W = 4, 512, 512
_N = _B * _H * _W
_R = 64
_NRB = _H // _R

_SEG_W = 1.0
_EDGE_W = 0.3
_ATT_W = 0.1


def _fold_lanes(v512):
    return (v512[:, 0:128] + v512[:, 128:256] + v512[:, 256:384]
            + v512[:, 384:512])


def _stage1_body(seg_ref, msk_ref, ein_ref, emk_ref,
                 nll_ref, mp_ref, stats_ref, edge_ref):
    b = pl.program_id(0)
    r = pl.program_id(1)

    x = seg_ref[0]
    t = msk_ref[0]
    ev = ein_ref[0, 0]
    et = emk_ref[0, 0]

    m = jnp.max(x, axis=0)
    ex = jnp.exp(x - m[None])
    lse = m + jnp.log(jnp.sum(ex, axis=0))

    cls = lax.broadcasted_iota(jnp.int32, (_NC, 1, 1), 0)
    eq = (t[None] == cls)
    lt = jnp.sum(jnp.where(eq, x, 0.0), axis=0)

    nll = lse - lt
    nll_ref[0] = nll
    mp_ref[0] = jnp.exp(lt - lse)

    validf = (ev > 0.8).astype(jnp.float32)
    eqf = eq.astype(jnp.float32) * validf[None]
    bins19 = jnp.sum(eqf, axis=(1, 2))
    s19 = jnp.sum(eqf * nll[None], axis=(1, 2))
    pad = jnp.zeros((_NC + 13,), jnp.float32)
    row0 = jnp.concatenate([bins19, pad[: 32 - _NC]])
    row1 = jnp.concatenate([s19, pad[: 32 - _NC]])
    upd = jnp.stack([row0, row1])[None]

    @pl.when(r == 0)
    def _():
        stats_ref[...] = jnp.zeros_like(stats_ref)
    stats_ref[...] += upd

    elem = (jnp.maximum(ev, 0.0) - ev * et
            + jnp.log(1.0 + jnp.exp(-jnp.abs(ev))))
    pos = _fold_lanes(jnp.sum(et, axis=0, keepdims=True))
    pose = _fold_lanes(jnp.sum(elem * et, axis=0, keepdims=True))
    nege = _fold_lanes(jnp.sum(elem * (1.0 - et), axis=0, keepdims=True))
    z = jnp.zeros((5, 128), jnp.float32)
    eupd = jnp.concatenate([pos, pose, nege, z], axis=0)

    @pl.when((b == 0) & (r == 0))
    def _():
        edge_ref[...] = jnp.zeros_like(edge_ref)
    edge_ref[...] += eupd


def _stage1(segin, edgein, segmask, edgemask):
    grid = (_B, _NRB)
    out_shapes = (
        jax.ShapeDtypeStruct((_B, _H, _W), jnp.float32),
        jax.ShapeDtypeStruct((_B, _H, _W), jnp.float32),
        jax.ShapeDtypeStruct((_B, 2, 32), jnp.float32),
        jax.ShapeDtypeStruct((8, 128), jnp.float32),
    )
    return pl.pallas_call(
        _stage1_body,
        grid=grid,
        in_specs=[
            pl.BlockSpec((1, _NC, _R, _W), lambda b, r: (b, 0, r, 0)),
            pl.BlockSpec((1, _R, _W), lambda b, r: (b, r, 0)),
            pl.BlockSpec((1, 1, _R, _W), lambda b, r: (b, 0, r, 0)),
            pl.BlockSpec((1, 1, _R, _W), lambda b, r: (b, 0, r, 0)),
        ],
        out_specs=(
            pl.BlockSpec((1, _R, _W), lambda b, r: (b, r, 0)),
            pl.BlockSpec((1, _R, _W), lambda b, r: (b, r, 0)),
            pl.BlockSpec((1, 2, 32), lambda b, r: (b, 0, 0)),
            pl.BlockSpec((8, 128), lambda b, r: (0, 0)),
        ),
        out_shape=out_shapes,
    )(segin, segmask, edgein, edgemask)


_NS = 16
_P = _N // _NS
_CHUNK = 8192
_NBIN = 1024
_ROW = 2048


def _sc_body(mp_hbm, nll_hbm, part_hbm,
             mp_v, buf_v, hist_v, loc_v, merged_v, row_v, prow_v, orow_v,
             sh_hist):
    wid = lax.axis_index("s")
    base = wid * _P
    lane = lax.iota(jnp.int32, 16)
    ones = jnp.ones((16,), jnp.float32)

    pltpu.sync_copy(mp_hbm.at[pl.ds(base, _P)], mp_v)

    def zero(ref, n):
        def zb(i, _):
            ref[pl.ds(i * 16, 16)] = jnp.zeros((16,), jnp.float32)
            return 0
        lax.fori_loop(0, n // 16, zb, 0)

    zero(prow_v, _ROW)
    zero(orow_v, _NBIN)

    def local_hist(shift, prefix, pshift):
        zero(hist_v, 16 * _NBIN)

        def sc(j, _):
            v = mp_v[pl.ds(j * 16, 16)]
            bits = plsc.bitcast(v, jnp.int32)
            bn = lax.shift_right_logical(bits, shift) & (_NBIN - 1)
            idx = lane * _NBIN + bn
            if pshift is None:
                plsc.addupdate_scatter(hist_v, [idx], ones)
            else:
                keep = lax.shift_right_logical(bits, pshift) == prefix
                plsc.addupdate_scatter(hist_v, [idx], ones, mask=keep)
            return 0
        lax.fori_loop(0, _P // 16, sc, 0)

        def rb(i, _):
            acc = jnp.zeros((16,), jnp.float32)
            for l in range(16):
                acc = acc + hist_v[pl.ds(l * _NBIN + i * 16, 16)]
            loc_v[pl.ds(i * 16, 16)] = acc
            return 0
        lax.fori_loop(0, _NBIN // 16, rb, 0)

    def merge_and_find(k_rem, pid):
        def cpub(i, a):
            prow_v[pl.ds(i * 16, 16)] = loc_v[pl.ds(i * 16, 16)]
            return a + jnp.sum(loc_v[pl.ds(i * 16, 16)])
        lsum = lax.fori_loop(0, _NBIN // 16, cpub, jnp.float32(0.0))
        ck = lsum + 1000.0 * (wid + 1).astype(jnp.float32) + 100000.0 * pid
        prow_v[pl.ds(_NBIN, 16)] = ck * ones
        pltpu.sync_copy(prow_v, sh_hist.at[wid])
        plsc.subcore_barrier()

        zero(merged_v, _NBIN)
        for l in range(_NS):
            def cond(c, l=l):
                return jnp.logical_and(jnp.logical_not(c[1]), c[0] < 64)

            def bd(c, l=l):
                i, _ok = c
                pltpu.sync_copy(sh_hist.at[l], row_v)

                def sm(j, a):
                    return a + jnp.sum(row_v[pl.ds(j * 16, 16)])
                s = lax.fori_loop(0, _NBIN // 16, sm, jnp.float32(0.0))
                exp = s + jnp.float32(1000.0 * (l + 1)) + 100000.0 * pid
                ckr = row_v[pl.ds(_NBIN, 16)]
                ok = jnp.logical_and(
                    jnp.abs(jnp.max(ckr) - jnp.min(ckr)) < 0.25,
                    jnp.abs(jnp.min(ckr) - exp) < 0.25)
                return i + 1, ok
            lax.while_loop(cond, bd, (jnp.int32(0), jnp.bool_(False)))

            def ac(i, _):
                merged_v[pl.ds(i * 16, 16)] = (
                    merged_v[pl.ds(i * 16, 16)] + row_v[pl.ds(i * 16, 16)])
                return 0
            lax.fori_loop(0, _NBIN // 16, ac, 0)
        plsc.subcore_barrier()

        def fb(j, carry):
            ccnt, found, rbelow = carry
            v = merged_v[pl.ds(j * 16, 16)]
            cs = plsc.cumsum(v)
            tot = jnp.max(cs)
            hit = (ccnt + cs) >= k_rem
            pc = jnp.max(plsc.all_reduce_population_count(hit))
            ln = 16 - pc
            below = jnp.sum(jnp.where(lane < ln, v, 0.0))
            is_hit = (found < 0) & (ccnt + tot >= k_rem)
            found = jnp.where(is_hit, j * 16 + ln, found)
            rbelow = jnp.where(is_hit, ccnt + below, rbelow)
            return ccnt + tot, found, rbelow
        _, fbin, rb_ = lax.fori_loop(
            0, _NBIN // 16, fb,
            (jnp.float32(0.0), jnp.int32(-1), jnp.float32(0.0)))
        return fbin, k_rem - rb_

    k0 = jnp.float32(_KEEP)
    local_hist(20, None, None)
    p1, k1 = merge_and_find(k0, jnp.float32(1.0))
    local_hist(10, p1, 20)
    p2, k2 = merge_and_find(k1, jnp.float32(2.0))
    p12 = p1 * 1024 + p2
    local_hist(0, p12, 10)
    p3, _k3 = merge_and_find(k2, jnp.float32(3.0))

    kth_bits = (p12 * 1024 + p3) * jnp.ones((16,), jnp.int32)
    kth = jnp.max(plsc.bitcast(kth_bits, jnp.float32))
    thr = jnp.maximum(kth, jnp.float32(_THR))

    sum_acc = jnp.zeros((16,), jnp.float32)
    cnt_acc = jnp.zeros((16,), jnp.float32)
    for s in range(_P // _CHUNK):
        pltpu.sync_copy(nll_hbm.at[pl.ds(base + s * _CHUNK, _CHUNK)], buf_v)

        def rd(j, carry, s=s):
            sa, ca = carry
            mpv = mp_v[pl.ds(s * _CHUNK + j * 16, 16)]
            nv = buf_v[pl.ds(j * 16, 16)]
            keep = mpv <= thr
            sa = sa + jnp.where(keep, nv, 0.0)
            ca = ca + jnp.where(keep, 1.0, 0.0)
            return sa, ca
        sum_acc, cnt_acc = lax.fori_loop(0, _CHUNK // 16, rd,
                                         (sum_acc, cnt_acc))

    orow_v[pl.ds(0, 16)] = sum_acc
    orow_v[pl.ds(16, 16)] = cnt_acc
    pltpu.sync_copy(orow_v, part_hbm.at[wid])


def _stage2(mp, nll):
    mesh = plsc.VectorSubcoreMesh(core_axis_name="c", subcore_axis_name="s",
                                  num_cores=1)
    f = pl.kernel(
        _sc_body,
        out_type=jax.ShapeDtypeStruct((_NS, _NBIN), jnp.float32),
        mesh=mesh,
        compiler_params=pltpu.CompilerParams(needs_layout_passes=False),
        scratch_types=[
            pltpu.VMEM((_P,), jnp.float32),
            pltpu.VMEM((_CHUNK,), jnp.float32),
            pltpu.VMEM((16 * _NBIN,), jnp.float32),
            pltpu.VMEM((_NBIN,), jnp.float32),
            pltpu.VMEM((_NBIN,), jnp.float32),
            pltpu.VMEM((_ROW,), jnp.float32),
            pltpu.VMEM((_ROW,), jnp.float32),
            pltpu.VMEM((_NBIN,), jnp.float32),
            pltpu.VMEM_SHARED((_NS, _ROW), jnp.float32),
        ],
    )
    return f(mp, nll)


def _stage3_body(part_ref, stats_ref, edge_ref, out_ref):
    part = part_ref[...]
    tsum = jnp.sum(part[:, 0:16])
    tcnt = jnp.sum(part[:, 16:32])
    seg = tsum / jnp.maximum(tcnt, 1.0)

    st = stats_ref[...]
    att = jnp.float32(0.0)
    for i in range(_B):
        b = st[i, 0]
        s = st[i, 1]
        tot = jnp.maximum(jnp.sum(b), 1.0)
        w = jnp.where(b != 0.0, 1.0 - b / tot, 0.0) + 1.0
        num = jnp.sum(w * s)
        den = jnp.sum(w * b)
        att = att + jnp.where(den > 0.0, num / jnp.maximum(den, 1e-8), 0.0)

    nf = jnp.float32(float(_N))
    pos_num = jnp.sum(edge_ref[0, :])
    pos_elem = jnp.sum(edge_ref[1, :])
    neg_elem = jnp.sum(edge_ref[2, :])
    neg_num = nf - pos_num
    edge = (neg_num / nf * pos_elem + pos_num / nf * neg_elem) / nf

    total = _SEG_W * seg + _EDGE_W * edge + _ATT_W * att
    out_ref[...] = jnp.full((8, 128), total, jnp.float32)


def _stage3(part, stats, edge):
    return pl.pallas_call(
        _stage3_body,
        out_shape=jax.ShapeDtypeStruct((8, 128), jnp.float32),
    )(part, stats, edge)


def kernel(segin, edgein, segmask, edgemask):
    nll, mp, stats, edge = _stage1(segin, edgein, segmask, edgemask)
    part = _stage2(mp.reshape(-1), nll.reshape(-1))
    out = _stage3(part, stats, edge)
    return out[0, 0]
